# baseline (speedup 1.0000x reference)
import functools

import jax
import jax.numpy as jnp
from jax import lax
from jax.experimental import pallas as pl
from jax.experimental.pallas import tpu as pltpu
from jax.experimental.pallas import tpu_sc as plsc

NU = 25000
NI = 25000
NN = NU + NI
D = 64
W = 32
WP = W // 2
E = 800000

CH = 128
NCH = 4
NST = 100
NCHT = NCH * NST
PT = CH * NCHT
EP = PT * 16
EROWS = EP // CH

NBG = 6
NBS = 3
LOOK = 4
NSTG = 4

ACC_ROWS = 50048
ZR = ACC_ROWS // 16
ZTAIL = ZR % CH
NOUT = NN - 15 * ZR

_PF = plsc.PackFormat.INTERLEAVED


def _layer_body(x_hbm, src_hbm, dst_hbm, val_hbm, out_hbm,
                src_st, dst_st, val_st, rows_g, rows_f, acc,
                semg, sems, semt):
    c = lax.axis_index("c")
    s = lax.axis_index("s")
    cbase = c * NN
    zbase = s * ZR

    def zrow_body(r, carry):
        for j in range(W // 16):
            rows_f[0, r, pl.ds(j * 16, 16)] = jnp.zeros((16,), jnp.float32)
        return carry
    lax.fori_loop(0, CH, zrow_body, 0)
    for k in range(ZR // CH):
        pltpu.sync_copy(rows_f.at[0], acc.at[pl.ds(zbase + k * CH, CH)])
    pltpu.sync_copy(rows_f.at[0, pl.ds(0, ZTAIL)],
                    acc.at[pl.ds(zbase + (ZR // CH) * CH, ZTAIL)])
    plsc.subcore_barrier()

    rbase = s * (PT // CH)

    def stage_fire(q):
        ps = lax.rem(q, NSTG)
        rb = rbase + q * NCH
        pltpu.async_copy(src_hbm.at[pl.ds(rb, NCH)], src_st.at[ps], semt)
        pltpu.async_copy(dst_hbm.at[pl.ds(rb, NCH)], dst_st.at[ps], semt)
        pltpu.async_copy(val_hbm.at[pl.ds(rb, NCH)], val_st.at[ps], semt)

    def stage_wait_adjust(q):
        ps = lax.rem(q, NSTG)
        rb = rbase
        pltpu.make_async_copy(src_hbm.at[pl.ds(rb, NCH)], src_st.at[ps], semt).wait()
        pltpu.make_async_copy(dst_hbm.at[pl.ds(rb, NCH)], dst_st.at[ps], semt).wait()
        pltpu.make_async_copy(val_hbm.at[pl.ds(rb, NCH)], val_st.at[ps], semt).wait()
        for kk in range(NCH):
            for g in range(CH // 16):
                sl = pl.ds(g * 16, 16)
                src_st[ps, kk, sl] = src_st[ps, kk, sl] + cbase

    def gather_fire(ps, kk, bg):
        pltpu.async_copy(x_hbm.at[src_st.at[ps, kk]], rows_g.at[bg],
                         semg.at[bg])

    def gather_wait(ps, kk, bg):
        pltpu.make_async_copy(x_hbm.at[src_st.at[ps, kk]], rows_g.at[bg],
                              semg.at[bg]).wait()

    def scatter_fire(ps, kk, bs):
        pltpu.async_copy(rows_f.at[bs], acc.at[dst_st.at[ps, kk]],
                         sems.at[bs], add=True)

    def scatter_wait(ps, kk, bs):
        pltpu.make_async_copy(rows_f.at[bs], acc.at[dst_st.at[ps, kk]],
                              sems.at[bs]).wait()

    stage_fire(0)
    stage_fire(1)
    stage_wait_adjust(0)
    stage_wait_adjust(1)
    stage_fire(2)
    for k0 in range(LOOK):
        gather_fire(k0 // NCH, k0 % NCH, k0 % NBG)

    def chunk_body(k, carry):
        q = k // NCH
        kk = k - q * NCH
        ps = lax.rem(q, NSTG)
        bg = lax.rem(k, NBG)
        bs = lax.rem(k, NBS)

        @pl.when(jnp.logical_and(jnp.logical_and(kk == 0, k > 0),
                                 q + 1 < NST))
        def _stage_ring():
            stage_wait_adjust(q + 1)

            @pl.when(q + 2 < NST)
            def _fire_stage():
                stage_fire(q + 2)

        @pl.when(k >= NBS)
        def _drain_scatter():
            ko = k - NBS
            qo = ko // NCH
            kko = ko - qo * NCH
            pso = lax.rem(qo, NSTG)
            scatter_wait(pso, kko, bs)

        gather_wait(ps, kk, bg)

        for g in range(CH // 16):
            vvals = val_st[ps, kk, pl.ds(g * 16, 16)]
            base = g * 16
            for i in range(16):
                vv = jnp.broadcast_to(vvals[i], (16,))
                rowi = rows_g[bg, base + i, pl.ds(0, WP)]
                lo_v = plsc.bitcast(rowi << 16, jnp.float32)
                hi_v = plsc.bitcast(rowi & (-65536), jnp.float32)
                rows_f[bs, base + i, pl.ds(0, 16)] = lo_v * vv
                rows_f[bs, base + i, pl.ds(16, 16)] = hi_v * vv

        scatter_fire(ps, kk, bs)

        kf = k + LOOK
        @pl.when(kf < NCHT)
        def _fire_next():
            qf = kf // NCH
            kkf = kf - qf * NCH
            psf = lax.rem(qf, NSTG)
            bgf = lax.rem(kf, NBG)
            gather_fire(psf, kkf, bgf)
        return carry

    lax.fori_loop(0, NCHT, chunk_body, 0)

    for kt in range(NCHT - NBS, NCHT):
        q = kt // NCH
        kk = kt - q * NCH
        scatter_wait(q % NSTG, kk, kt % NBS)

    plsc.subcore_barrier()

    nch_out = ZR // CH + 1
    sizes = [CH] * (ZR // CH) + [ZTAIL]
    obase = cbase + zbase
    last = (s == 15)

    def din_fire(k):
        p = k % 2
        pltpu.async_copy(acc.at[pl.ds(zbase + k * CH, sizes[k])],
                         rows_f.at[p, pl.ds(0, sizes[k])], semg.at[p])

    def din_wait(k):
        p = k % 2
        pltpu.make_async_copy(acc.at[pl.ds(zbase + k * CH, sizes[k])],
                              rows_f.at[p, pl.ds(0, sizes[k])],
                              semg.at[p]).wait()

    def dout_fire(k, n):
        p = k % 2
        pltpu.async_copy(rows_g.at[p, pl.ds(0, n)],
                         out_hbm.at[pl.ds(obase + k * CH, n)],
                         sems.at[p])

    def dout_wait(k, n):
        p = k % 2
        pltpu.make_async_copy(rows_g.at[p, pl.ds(0, n)],
                              out_hbm.at[pl.ds(obase + k * CH, n)],
                              sems.at[p]).wait()

    din_fire(0)
    for k in range(nch_out):
        p = k % 2
        din_wait(k)
        if k + 1 < nch_out:
            din_fire(k + 1)
        if k >= 2:
            dout_wait(k - 2, sizes[k - 2])

        def pk_body(r, carry):
            lo_u = plsc.bitcast(rows_f[p, r, pl.ds(0, 16)], jnp.int32)
            hi_u = plsc.bitcast(rows_f[p, r, pl.ds(16, 16)], jnp.int32)
            lo_r = ((lo_u + 32767 + ((lo_u >> 16) & 1)) >> 16) & 65535
            hi_r = (hi_u + 32767 + ((hi_u >> 16) & 1)) & (-65536)
            rows_g[p, r, pl.ds(0, WP)] = lo_r | hi_r
            return carry
        lax.fori_loop(0, sizes[k], pk_body, 0)

        if k == nch_out - 1:
            @pl.when(jnp.logical_not(last))
            def _dout_tail():
                dout_fire(k, ZTAIL)

            @pl.when(last)
            def _dout_tail15():
                dout_fire(k, NOUT - 24 * CH)
        else:
            dout_fire(k, sizes[k])

    dout_wait(nch_out - 2, sizes[nch_out - 2])

    @pl.when(jnp.logical_not(last))
    def _drain_out():
        dout_wait(nch_out - 1, ZTAIL)

    @pl.when(last)
    def _drain_out15():
        dout_wait(nch_out - 1, NOUT - 24 * CH)


_layer = functools.partial(
    pl.kernel,
    out_type=jax.ShapeDtypeStruct((2 * NN, WP), jnp.int32),
    mesh=plsc.VectorSubcoreMesh(core_axis_name="c", subcore_axis_name="s",
                                num_cores=2, num_subcores=16),
    compiler_params=pltpu.CompilerParams(use_tc_tiling_on_sc=False,
                                         needs_layout_passes=False),
    scratch_types=[
        pltpu.VMEM((NSTG, NCH, CH), jnp.int32),
        pltpu.VMEM((NSTG, NCH, CH), jnp.int32),
        pltpu.VMEM((NSTG, NCH, CH), jnp.float32),
        pltpu.VMEM((NBG, CH, WP), jnp.int32),
        pltpu.VMEM((NBS, CH, W), jnp.float32),
        pltpu.VMEM_SHARED((ACC_ROWS, W), jnp.float32),
        pltpu.SemaphoreType.DMA((NBG,)),
        pltpu.SemaphoreType.DMA((NBS,)),
        pltpu.SemaphoreType.DMA,
    ],
)(_layer_body)


def _mean_body(l0_ref, h0_ref, a_ref, b_ref, c_ref, olo_ref, ohi_ref):
    def dec_lo(x):
        return jax.lax.bitcast_convert_type(x << 16, jnp.float32)

    def dec_hi(x):
        return jax.lax.bitcast_convert_type(x & (-65536), jnp.float32)

    a, b, c = a_ref[...], b_ref[...], c_ref[...]
    olo_ref[...] = (l0_ref[...] + dec_lo(a) + dec_lo(b) + dec_lo(c)) * 0.25
    ohi_ref[...] = (h0_ref[...] + dec_hi(a) + dec_hi(b) + dec_hi(c)) * 0.25


def _mean4(l0, h0, a, b, c):
    blk = (400, WP)
    spec = pl.BlockSpec(blk, lambda i: (i, 0))
    return pl.pallas_call(
        _mean_body,
        grid=(2 * NN // blk[0],),
        in_specs=[spec] * 5,
        out_specs=(spec, spec),
        out_shape=(jax.ShapeDtypeStruct((2 * NN, WP), jnp.float32),
                   jax.ShapeDtypeStruct((2 * NN, WP), jnp.float32)),
    )(l0, h0, a, b, c)


def kernel(adj_indices, adj_values, user_emb, item_emb):
    x0 = jnp.concatenate([user_emb, item_emb], axis=0)
    x0s = jnp.concatenate([x0[:, :W], x0[:, W:]], axis=0)
    x0lo, x0hi = x0s[:, :WP], x0s[:, WP:]
    bl = lax.bitcast_convert_type(x0lo.astype(jnp.bfloat16), jnp.uint16)
    bh = lax.bitcast_convert_type(x0hi.astype(jnp.bfloat16), jnp.uint16)
    x0i = lax.bitcast_convert_type(
        bl.astype(jnp.uint32) | (bh.astype(jnp.uint32) << 16), jnp.int32)

    dst = adj_indices[0].astype(jnp.int32)
    src = adj_indices[1].astype(jnp.int32)
    pad = EP - E
    src2 = jnp.concatenate([src, jnp.zeros((pad,), jnp.int32)]).reshape(EROWS, CH)
    dst2 = jnp.concatenate([dst, jnp.full((pad,), NN, jnp.int32)]).reshape(EROWS, CH)
    val2 = jnp.concatenate([adj_values.astype(jnp.float32),
                            jnp.zeros((pad,), jnp.float32)]).reshape(EROWS, CH)

    x1 = _layer(x0i, src2, dst2, val2)
    x2 = _layer(x1, src2, dst2, val2)
    x3 = _layer(x2, src2, dst2, val2)
    mlo, mhi = _mean4(x0lo, x0hi, x1, x2, x3)
    out = jnp.concatenate([mlo[:NN], mhi[:NN], mlo[NN:], mhi[NN:]], axis=1)
    return out[:NU], out[NU:]
